# baseline (speedup 1.0000x reference)
import functools
import math

import jax
import jax.numpy as jnp
from jax import lax
from jax.experimental import pallas as pl
from jax.experimental.pallas import tpu as pltpu
from jax.experimental.pallas import tpu_sc as plsc

D_MODEL = 64
SCALE = math.sqrt(D_MODEL)

_info = plsc.get_sparse_core_info()
_NC = _info.num_cores
_NS = _info.num_subcores
_NW = _NC * _NS
_L = _info.num_lanes

_AD = 4
_BD = 2


@functools.lru_cache(maxsize=None)
def _build(R: int, C: int, V: int):
    assert R % _NW == 0 and C % _AD == 0, (R, C)
    bw = R // _NW
    n_steps = C // _AD

    mesh = plsc.VectorSubcoreMesh(core_axis_name="c", subcore_axis_name="s")

    @functools.partial(
        pl.kernel,
        mesh=mesh,
        out_type=jax.ShapeDtypeStruct((R, C, D_MODEL), jnp.float32),
        scratch_types=[pltpu.VMEM((C, bw), jnp.int32)]
        + [pltpu.VMEM((bw, 2 * D_MODEL), jnp.float32)] * _AD
        + [pltpu.VMEM((bw, 1, D_MODEL), jnp.float32)] * _BD
        + [pltpu.SemaphoreType.DMA] * (_AD + _BD),
    )
    def emb(xt_hbm, table_hbm, out_hbm, idx_v, a0, a1, a2, a3, b0, b1,
            sg0, sg1, sg2, sg3, ss0, ss1):
        abuf = [a0, a1, a2, a3]
        bbuf = [b0, b1]
        sg = [sg0, sg1, sg2, sg3]
        ss = [ss0, ss1]

        wid = lax.axis_index("s") * _NC + lax.axis_index("c")
        col0 = wid * bw
        pltpu.sync_copy(xt_hbm.at[:, pl.ds(col0, bw)], idx_v)

        def fire_gather(g, buf, sem):
            pltpu.async_copy(table_hbm.at[idx_v.at[g, :]], buf, sem)

        def drain_gather(g, buf, sem):
            pltpu.make_async_copy(
                table_hbm.at[idx_v.at[g, :]], buf, sem
            ).wait()

        for j in range(_AD):
            fire_gather(j, abuf[j], sg[j])

        @pl.loop(0, n_steps)
        def step(s):
            for j in range(_AD):
                g = s * _AD + j
                drain_gather(g, abuf[j], sg[j])

                def wait_scatter(g=g, j=j):
                    pltpu.make_async_copy(
                        bbuf[j % _BD],
                        out_hbm.at[pl.ds(col0, bw), pl.ds(g - _BD, 1)],
                        ss[j % _BD],
                    ).wait()

                if j < _BD:
                    pl.when(s > 0)(wait_scatter)
                else:
                    wait_scatter()

                src = abuf[j]
                dst = bbuf[j % _BD]

                @plsc.parallel_loop(0, bw, 1, unroll=8)
                def pack_row(r, src=src, dst=dst):
                    for c in range(D_MODEL // _L):
                        sl = pl.ds(c * _L, _L)
                        dst[r, 0, sl] = src[r, sl]

                pltpu.async_copy(
                    dst, out_hbm.at[pl.ds(col0, bw), pl.ds(g, 1)], ss[j % _BD]
                )

                def refire(g=g, j=j):
                    fire_gather(g + _AD, abuf[j], sg[j])

                pl.when(s < n_steps - 1)(refire)

        for j in range(_BD):
            pltpu.make_async_copy(
                bbuf[j],
                out_hbm.at[pl.ds(col0, bw), pl.ds(C - _BD + j, 1)],
                ss[j],
            ).wait()

    return emb


def kernel(x, table):
    R, C = x.shape
    V = table.shape[0]
    xt = x.astype(jnp.int32).T
    tpad = jnp.pad(table * jnp.float32(SCALE), ((0, 0), (0, D_MODEL)))
    return _build(R, C, V)(xt, tpad)


# speedup vs baseline: 1.1918x; 1.1918x over previous
import functools
import math

import jax
import jax.numpy as jnp
from jax import lax
from jax.experimental import pallas as pl
from jax.experimental.pallas import tpu as pltpu
from jax.experimental.pallas import tpu_sc as plsc

D_MODEL = 64
SCALE = math.sqrt(D_MODEL)

_info = plsc.get_sparse_core_info()
_NC = _info.num_cores
_NS = _info.num_subcores
_NW = _NC * _NS
_L = _info.num_lanes

_AD = 4
_BD = 2


@functools.lru_cache(maxsize=None)
def _build(R: int, C: int, V: int):
    assert R % _NW == 0 and C % _AD == 0, (R, C)
    bw = R // _NW
    n_steps = C // _AD

    mesh = plsc.VectorSubcoreMesh(core_axis_name="c", subcore_axis_name="s")

    @functools.partial(
        pl.kernel,
        mesh=mesh,
        out_type=jax.ShapeDtypeStruct((R, C, D_MODEL), jnp.float32),
        scratch_types=[pltpu.VMEM((C, bw), jnp.int32)]
        + [pltpu.VMEM((bw,), jnp.int32)] * _AD
        + [pltpu.VMEM((bw, 2 * D_MODEL), jnp.float32)] * _AD
        + [pltpu.VMEM((bw, 1, D_MODEL), jnp.float32)] * _BD
        + [pltpu.SemaphoreType.DMA] * (_AD + _BD),
    )
    def emb(xt_hbm, table_hbm, out_hbm, idx_v, h0, h1, h2, h3,
            a0, a1, a2, a3, b0, b1, sg0, sg1, sg2, sg3, ss0, ss1):
        hbuf = [h0, h1, h2, h3]
        abuf = [a0, a1, a2, a3]
        bbuf = [b0, b1]
        sg = [sg0, sg1, sg2, sg3]
        ss = [ss0, ss1]

        wid = lax.axis_index("s") * _NC + lax.axis_index("c")
        col0 = wid * bw
        pltpu.sync_copy(xt_hbm.at[:, pl.ds(col0, bw)], idx_v)

        def fire_gather(g, hb, buf, sem):
            for v in range(bw // _L):
                sl = pl.ds(v * _L, _L)
                hb[sl] = jax.lax.shift_right_logical(idx_v[g, sl], 1)
            pltpu.async_copy(table_hbm.at[hb], buf, sem)

        def drain_gather(hb, buf, sem):
            pltpu.make_async_copy(table_hbm.at[hb], buf, sem).wait()

        for j in range(_AD):
            fire_gather(j, hbuf[j], abuf[j], sg[j])

        @pl.loop(0, n_steps)
        def step(s):
            for j in range(_AD):
                g = s * _AD + j
                drain_gather(hbuf[j], abuf[j], sg[j])

                def wait_scatter(g=g, j=j):
                    pltpu.make_async_copy(
                        bbuf[j % _BD],
                        out_hbm.at[pl.ds(col0, bw), pl.ds(g - _BD, 1)],
                        ss[j % _BD],
                    ).wait()

                if j < _BD:
                    pl.when(s > 0)(wait_scatter)
                else:
                    wait_scatter()

                src = abuf[j]
                dst = bbuf[j % _BD]

                @plsc.parallel_loop(0, bw // _L, 1)
                def scale_group(t, g=g, src=src, dst=dst):
                    offv = (idx_v[g, pl.ds(t * _L, _L)] & 1) * D_MODEL
                    for l in range(_L):
                        r = t * _L + l
                        off = offv[l]
                        for c in range(D_MODEL // _L):
                            dst[r, 0, pl.ds(c * _L, _L)] = (
                                src[r, pl.ds(off + c * _L, _L)] * SCALE
                            )

                pltpu.async_copy(
                    dst, out_hbm.at[pl.ds(col0, bw), pl.ds(g, 1)], ss[j % _BD]
                )

                def refire(g=g, j=j):
                    fire_gather(g + _AD, hbuf[j], abuf[j], sg[j])

                pl.when(s < n_steps - 1)(refire)

        for j in range(_BD):
            pltpu.make_async_copy(
                bbuf[j],
                out_hbm.at[pl.ds(col0, bw), pl.ds(C - _BD + j, 1)],
                ss[j],
            ).wait()

    return emb


def kernel(x, table):
    R, C = x.shape
    V = table.shape[0]
    xt = x.astype(jnp.int32).T
    table2 = table.reshape(V // 2, 2 * D_MODEL)
    return _build(R, C, V)(xt, table2)
